# Initial kernel scaffold; baseline (speedup 1.0000x reference)
import jax
import jax.numpy as jnp
from jax import lax
from jax.experimental import pallas as pl
from jax.experimental.pallas import tpu as pltpu
from jax.experimental.pallas import tpu_sc as plsc

_NUM_USERS = 10000
_NUM_ITEMS = 40000
_DIM = 64
_N = _NUM_USERS + _NUM_ITEMS
_E = 800000

_NC, _NS, _L = 2, 16, 16
_HALF = 25600
_N_PAD = _NC * _HALF
_K = 128
_NB = 4
_EPT = 51200
_E_PAD = _EPT * _NS
_CPT = _EPT // _K
_BLK = 16
_NBLK = _CPT // _BLK
_ROWS_PT = _HALF // _NS


def _zero_acc(s, acc, zbuf):
    zero16 = jnp.zeros((_L,), jnp.float32)

    def zrow(i, _):
        for k4 in range(_DIM // _L):
            zbuf[i, pl.ds(k4 * _L, _L)] = zero16
        return 0

    lax.fori_loop(0, 64, zrow, 0)

    def zacc(b, _):
        pltpu.sync_copy(zbuf, acc.at[pl.ds(s * _ROWS_PT + b * 64, 64)])
        return 0

    lax.fori_loop(0, _ROWS_PT // 64, zacc, 0)


def _edge_pass(c, s, emat, col2, dst2, w2, acc,
               colbuf, rows, dstbuf, wbuf, sidx, gsem, ssem):
    base = c * _HALF
    pltpu.sync_copy(col2.at[pl.ds(s * _CPT, _CPT)], colbuf)
    for t in range(2):
        pltpu.async_copy(emat.at[colbuf.at[t]], rows.at[t], gsem.at[t])

    def blk_body(blk, _):
        eoff = s * _CPT + blk * _BLK
        pltpu.sync_copy(dst2.at[pl.ds(eoff, _BLK)], dstbuf)
        pltpu.sync_copy(w2.at[pl.ds(eoff, _BLK)], wbuf)

        def rnd_body(r, _):
            for bb in range(_NB):
                j = r * _NB + bb
                t = blk * _BLK + j
                bb2 = (bb + 2) % _NB
                pltpu.make_async_copy(
                    emat.at[colbuf.at[t]], rows.at[bb], gsem.at[bb]).wait()

                @pl.when(t >= 2)
                def _():
                    pltpu.make_async_copy(
                        rows.at[bb2], acc.at[sidx.at[bb2]],
                        ssem.at[bb2]).wait()

                @pl.when(t + 2 < _CPT)
                def _():
                    pltpu.async_copy(
                        emat.at[colbuf.at[t + 2]], rows.at[bb2],
                        gsem.at[bb2])

                for g in range(_K // _L):
                    d = dstbuf[j, pl.ds(g * _L, _L)]
                    lo = d - base
                    ok = (lo >= 0) & (lo < _HALF)
                    sidx[bb, pl.ds(g * _L, _L)] = jnp.where(ok, lo, _HALF)

                def mul_body(e, _):
                    wv = wbuf[j, e]
                    for k4 in range(_DIM // _L):
                        sl = pl.ds(k4 * _L, _L)
                        rows[bb, e, sl] = rows[bb, e, sl] * wv
                    return 0

                lax.fori_loop(0, _K, mul_body, 0)

                pltpu.async_copy(rows.at[bb], acc.at[sidx.at[bb]],
                                 ssem.at[bb], add=True)
            return 0

        lax.fori_loop(0, _BLK // _NB, rnd_body, 0)
        return 0

    lax.fori_loop(0, _NBLK, blk_body, 0)
    for bb in (2, 3):
        pltpu.make_async_copy(rows.at[bb], acc.at[sidx.at[bb]],
                              ssem.at[bb]).wait()


def _prop_body(emat, col2, dst2, w2, out, acc,
               colbuf, rows, dstbuf, wbuf, sidx, zbuf, cb1, cb2,
               gsem, ssem):
    c = lax.axis_index("c")
    s = lax.axis_index("s")
    _zero_acc(s, acc, zbuf)
    plsc.subcore_barrier()
    _edge_pass(c, s, emat, col2, dst2, w2, acc,
               colbuf, rows, dstbuf, wbuf, sidx, gsem, ssem)
    plsc.subcore_barrier()
    pltpu.sync_copy(acc.at[pl.ds(s * _ROWS_PT, _ROWS_PT)],
                    out.at[pl.ds(c * _HALF + s * _ROWS_PT, _ROWS_PT)])


def _combine_body(emat, col2, dst2, w2, e0, out, acc,
                  colbuf, rows, dstbuf, wbuf, sidx, zbuf, cb1, cb2,
                  gsem, ssem):
    c = lax.axis_index("c")
    s = lax.axis_index("s")
    _zero_acc(s, acc, zbuf)
    plsc.subcore_barrier()
    _edge_pass(c, s, emat, col2, dst2, w2, acc,
               colbuf, rows, dstbuf, wbuf, sidx, gsem, ssem)
    plsc.subcore_barrier()
    r0 = c * _HALF + s * _ROWS_PT
    third = jnp.float32(1.0 / 3.0)

    def cb(b, _):
        g0 = r0 + b * 64
        l0 = s * _ROWS_PT + b * 64
        pltpu.sync_copy(e0.at[pl.ds(g0, 64)], zbuf)
        pltpu.sync_copy(emat.at[pl.ds(g0, 64)], cb1)
        pltpu.sync_copy(acc.at[pl.ds(l0, 64)], cb2)

        def rowf(i, _):
            for k4 in range(_DIM // _L):
                sl = pl.ds(k4 * _L, _L)
                zbuf[i, sl] = (zbuf[i, sl] + cb1[i, sl] + cb2[i, sl]) * third
            return 0

        lax.fori_loop(0, 64, rowf, 0)
        pltpu.sync_copy(zbuf, out.at[pl.ds(g0, 64)])
        return 0

    lax.fori_loop(0, _ROWS_PT // 64, cb, 0)


_SCRATCH = [
    pltpu.VMEM_SHARED((_HALF + 8, _DIM), jnp.float32),
    pltpu.VMEM((_CPT, _K), jnp.int32),
    pltpu.VMEM((_NB, _K, _DIM), jnp.float32),
    pltpu.VMEM((_BLK, _K), jnp.int32),
    pltpu.VMEM((_BLK, _K), jnp.float32),
    pltpu.VMEM((_NB, _K), jnp.int32),
    pltpu.VMEM((64, _DIM), jnp.float32),
    pltpu.VMEM((64, _DIM), jnp.float32),
    pltpu.VMEM((64, _DIM), jnp.float32),
    pltpu.SemaphoreType.DMA((_NB,)),
    pltpu.SemaphoreType.DMA((_NB,)),
]

_MESH = plsc.VectorSubcoreMesh(core_axis_name="c", subcore_axis_name="s")
_OUT = jax.ShapeDtypeStruct((_N_PAD, _DIM), jnp.float32)

_prop = pl.kernel(_prop_body, out_type=_OUT, mesh=_MESH,
                  scratch_types=_SCRATCH, name="lightgcn_prop")
_combine = pl.kernel(_combine_body, out_type=_OUT, mesh=_MESH,
                     scratch_types=_SCRATCH, name="lightgcn_prop_combine")


def kernel(edge_index, edge_weight, user_emb, item_emb):
    emat0 = jnp.concatenate(
        [user_emb, item_emb,
         jnp.zeros((_N_PAD - _N, _DIM), jnp.float32)], axis=0)
    dst = edge_index[0]
    col = edge_index[1]
    padn = _E_PAD - _E
    col_p = jnp.concatenate([col, jnp.zeros((padn,), col.dtype)])
    dst_p = jnp.concatenate([dst, jnp.full((padn,), _N_PAD - 1, dst.dtype)])
    w_p = jnp.concatenate([edge_weight, jnp.zeros((padn,), jnp.float32)])
    col2 = col_p.reshape(_E_PAD // _K, _K).astype(jnp.int32)
    dst2 = dst_p.reshape(_E_PAD // _K, _K).astype(jnp.int32)
    w2 = w_p.reshape(_E_PAD // _K, _K)
    e1 = _prop(emat0, col2, dst2, w2)
    o = _combine(e1, col2, dst2, w2, emat0)
    return o[:_NUM_USERS], o[_NUM_USERS:_N]


# speedup vs baseline: 4.6016x; 4.6016x over previous
import jax
import jax.numpy as jnp
from jax import lax
from jax.experimental import pallas as pl
from jax.experimental.pallas import tpu as pltpu
from jax.experimental.pallas import tpu_sc as plsc

_NUM_USERS = 10000
_NUM_ITEMS = 40000
_DIM = 64
_N = _NUM_USERS + _NUM_ITEMS
_E = 800000

_NC, _NS, _L = 2, 16, 16
_HALF = 25344
_N_PAD = _NC * _HALF
_K = 64
_NR = 6
_NE = 8
_EPT = 50048
_E_PAD = _EPT * _NS
_CPT = _EPT // _K
_ROWS_PT = _HALF // _NS
_WCH = 8
_NWCH = _ROWS_PT // _WCH


def _zero_acc(s, acc, zbuf):
    zero16 = jnp.zeros((_L,), jnp.float32)
    for i in range(_WCH):
        for k4 in range(_DIM // _L):
            zbuf[i, pl.ds(k4 * _L, _L)] = zero16

    def zacc(b, _):
        pltpu.sync_copy(zbuf, acc.at[pl.ds(s * _ROWS_PT + b * _WCH, _WCH)])
        return 0

    lax.fori_loop(0, _NWCH, zacc, 0)


def _edge_pass(c, s, emat, edata, acc, ering, rows, sidx, esem, gsem, ssem):
    base = c * _HALF
    crow0 = s * _CPT

    for t in range(_NE - 2):
        pltpu.async_copy(edata.at[crow0 + t], ering.at[t], esem.at[t])
    for t in range(3):
        pltpu.make_async_copy(edata.at[crow0 + t], ering.at[t],
                              esem.at[t]).wait()
        pltpu.async_copy(emat.at[ering.at[t, 0]], rows.at[t], gsem.at[t])

    def chunk_body(t, _):
        bb = lax.rem(t, _NR)
        e8 = lax.rem(t, _NE)
        s3 = lax.rem(t + 3, _NR)
        e3 = lax.rem(t + 3, _NE)
        e6 = lax.rem(t + 6, _NE)

        @pl.when(t + 6 < _CPT)
        def _():
            pltpu.async_copy(edata.at[crow0 + t + 6], ering.at[e6],
                             esem.at[e6])

        @pl.when(t >= 3)
        def _():
            pltpu.make_async_copy(rows.at[s3], acc.at[sidx.at[s3]],
                                  ssem.at[s3]).wait()

        @pl.when(t + 3 < _CPT)
        def _():
            pltpu.make_async_copy(edata.at[crow0 + t + 3], ering.at[e3],
                                  esem.at[e3]).wait()
            pltpu.async_copy(emat.at[ering.at[e3, 0]], rows.at[s3],
                             gsem.at[s3])

        pltpu.make_async_copy(emat.at[ering.at[e8, 0]], rows.at[bb],
                              gsem.at[bb]).wait()

        for g in range(_K // _L):
            d = ering[e8, 1, pl.ds(g * _L, _L)]
            lo = d - base
            ok = (lo >= 0) & (lo < _HALF)
            sidx[bb, pl.ds(g * _L, _L)] = jnp.where(ok, lo, _HALF)

        for g in range(_K // _L):
            w16 = plsc.bitcast(ering[e8, 2, pl.ds(g * _L, _L)], jnp.float32)
            for e in range(_L):
                wv = w16[e]
                er = g * _L + e
                for k4 in range(_DIM // _L):
                    sl = pl.ds(k4 * _L, _L)
                    rows[bb, er, sl] = rows[bb, er, sl] * wv

        pltpu.async_copy(rows.at[bb], acc.at[sidx.at[bb]], ssem.at[bb],
                         add=True)
        return 0

    lax.fori_loop(0, _CPT, chunk_body, 0)
    for t in range(_CPT - 3, _CPT):
        bb = t % _NR
        pltpu.make_async_copy(rows.at[bb], acc.at[sidx.at[bb]],
                              ssem.at[bb]).wait()


def _prop_body(emat, edata, out, acc, ering, rows, sidx, zbuf, cb1, cb2,
               esem, gsem, ssem):
    c = lax.axis_index("c")
    s = lax.axis_index("s")
    _zero_acc(s, acc, zbuf)
    plsc.subcore_barrier()
    _edge_pass(c, s, emat, edata, acc, ering, rows, sidx, esem, gsem, ssem)
    plsc.subcore_barrier()
    pltpu.sync_copy(acc.at[pl.ds(s * _ROWS_PT, _ROWS_PT)],
                    out.at[pl.ds(c * _HALF + s * _ROWS_PT, _ROWS_PT)])


def _combine_body(emat, edata, e0, out, acc, ering, rows, sidx, zbuf,
                  cb1, cb2, esem, gsem, ssem):
    c = lax.axis_index("c")
    s = lax.axis_index("s")
    _zero_acc(s, acc, zbuf)
    plsc.subcore_barrier()
    _edge_pass(c, s, emat, edata, acc, ering, rows, sidx, esem, gsem, ssem)
    plsc.subcore_barrier()
    r0 = c * _HALF + s * _ROWS_PT
    third = jnp.float32(1.0 / 3.0)

    def cb(b, _):
        g0 = r0 + b * _WCH
        l0 = s * _ROWS_PT + b * _WCH
        pltpu.sync_copy(e0.at[pl.ds(g0, _WCH)], zbuf)
        pltpu.sync_copy(emat.at[pl.ds(g0, _WCH)], cb1)
        pltpu.sync_copy(acc.at[pl.ds(l0, _WCH)], cb2)
        for i in range(_WCH):
            for k4 in range(_DIM // _L):
                sl = pl.ds(k4 * _L, _L)
                zbuf[i, sl] = (zbuf[i, sl] + cb1[i, sl] + cb2[i, sl]) * third
        pltpu.sync_copy(zbuf, out.at[pl.ds(g0, _WCH)])
        return 0

    lax.fori_loop(0, _NWCH, cb, 0)


_SCRATCH = [
    pltpu.VMEM_SHARED((_HALF + 8, _DIM), jnp.float32),
    pltpu.VMEM((_NE, 3, _K), jnp.int32),
    pltpu.VMEM((_NR, _K, _DIM), jnp.float32),
    pltpu.VMEM((_NR, _K), jnp.int32),
    pltpu.VMEM((_WCH, _DIM), jnp.float32),
    pltpu.VMEM((_WCH, _DIM), jnp.float32),
    pltpu.VMEM((_WCH, _DIM), jnp.float32),
    pltpu.SemaphoreType.DMA((_NE,)),
    pltpu.SemaphoreType.DMA((_NR,)),
    pltpu.SemaphoreType.DMA((_NR,)),
]

_MESH = plsc.VectorSubcoreMesh(core_axis_name="c", subcore_axis_name="s")
_OUT = jax.ShapeDtypeStruct((_N_PAD, _DIM), jnp.float32)
_PARAMS = pltpu.CompilerParams(use_tc_tiling_on_sc=False,
                               needs_layout_passes=False)

_prop = pl.kernel(_prop_body, out_type=_OUT, mesh=_MESH,
                  scratch_types=_SCRATCH, compiler_params=_PARAMS,
                  name="lightgcn_prop")
_combine = pl.kernel(_combine_body, out_type=_OUT, mesh=_MESH,
                     scratch_types=_SCRATCH, compiler_params=_PARAMS,
                     name="lightgcn_prop_combine")


def kernel(edge_index, edge_weight, user_emb, item_emb):
    emat0 = jnp.concatenate(
        [user_emb, item_emb,
         jnp.zeros((_N_PAD - _N, _DIM), jnp.float32)], axis=0)
    dst = edge_index[0].astype(jnp.int32)
    col = edge_index[1].astype(jnp.int32)
    padn = _E_PAD - _E
    col_p = jnp.concatenate([col, jnp.zeros((padn,), jnp.int32)])
    dst_p = jnp.concatenate([dst, jnp.full((padn,), _N_PAD - 1, jnp.int32)])
    w_p = jnp.concatenate([edge_weight, jnp.zeros((padn,), jnp.float32)])
    edata = jnp.stack(
        [col_p.reshape(_E_PAD // _K, _K),
         dst_p.reshape(_E_PAD // _K, _K),
         jax.lax.bitcast_convert_type(w_p, jnp.int32).reshape(
             _E_PAD // _K, _K)], axis=1)
    e1 = _prop(emat0, edata)
    o = _combine(e1, edata, emat0)
    return o[:_NUM_USERS], o[_NUM_USERS:_N]


# speedup vs baseline: 4.6635x; 1.0134x over previous
import jax
import jax.numpy as jnp
from jax import lax
from jax.experimental import pallas as pl
from jax.experimental.pallas import tpu as pltpu
from jax.experimental.pallas import tpu_sc as plsc

_NUM_USERS = 10000
_NUM_ITEMS = 40000
_DIM = 64
_N = _NUM_USERS + _NUM_ITEMS
_E = 800000

_NC, _NS, _L = 2, 16, 16
_HALF = 25344
_N_PAD = _NC * _HALF
_K = 64
_NR = 6
_NE = 8
_EPT = 50048
_E_PAD = _EPT * _NS
_CPT = _EPT // _K
_ROWS_PT = _HALF // _NS
_WCH = 8
_NWCH = _ROWS_PT // _WCH


def _zero_acc(s, acc, cbun, zsem):
    zero16 = jnp.zeros((_L,), jnp.float32)
    for i in range(_WCH):
        for k4 in range(_DIM // _L):
            cbun[0, 0, i, pl.ds(k4 * _L, _L)] = zero16
    zbuf = cbun.at[0, 0]

    def zacc(b, _):
        @pl.when(b >= 8)
        def _():
            pltpu.make_async_copy(zbuf, acc.at[pl.ds(0, _WCH)], zsem).wait()

        pltpu.async_copy(zbuf, acc.at[pl.ds(s * _ROWS_PT + b * _WCH, _WCH)],
                         zsem)
        return 0

    lax.fori_loop(0, _NWCH, zacc, 0)

    def zdrain(b, _):
        pltpu.make_async_copy(zbuf, acc.at[pl.ds(0, _WCH)], zsem).wait()
        return 0

    lax.fori_loop(0, min(8, _NWCH), zdrain, 0)


def _edge_pass(c, s, emat, edata, acc, ering, rows, sidx, esem, gsem, ssem):
    base = c * _HALF
    crow0 = s * _CPT

    for t in range(_NE - 2):
        pltpu.async_copy(edata.at[crow0 + t], ering.at[t], esem.at[t])
    for t in range(3):
        pltpu.make_async_copy(edata.at[crow0 + t], ering.at[t],
                              esem.at[t]).wait()
        pltpu.async_copy(emat.at[ering.at[t, 0]], rows.at[t], gsem.at[t])

    def chunk_body(t, _):
        bb = lax.rem(t, _NR)
        e8 = lax.rem(t, _NE)
        s3 = lax.rem(t + 3, _NR)
        e3 = lax.rem(t + 3, _NE)
        e6 = lax.rem(t + 6, _NE)

        @pl.when(t + 6 < _CPT)
        def _():
            pltpu.async_copy(edata.at[crow0 + t + 6], ering.at[e6],
                             esem.at[e6])

        @pl.when(t >= 3)
        def _():
            pltpu.make_async_copy(rows.at[s3], acc.at[sidx.at[s3]],
                                  ssem.at[s3]).wait()

        @pl.when(t + 3 < _CPT)
        def _():
            pltpu.make_async_copy(edata.at[crow0 + t + 3], ering.at[e3],
                                  esem.at[e3]).wait()
            pltpu.async_copy(emat.at[ering.at[e3, 0]], rows.at[s3],
                             gsem.at[s3])

        pltpu.make_async_copy(emat.at[ering.at[e8, 0]], rows.at[bb],
                              gsem.at[bb]).wait()

        for g in range(_K // _L):
            d = ering[e8, 1, pl.ds(g * _L, _L)]
            lo = d - base
            ok = (lo >= 0) & (lo < _HALF)
            sidx[bb, pl.ds(g * _L, _L)] = jnp.where(ok, lo, _HALF)

        for g in range(_K // _L):
            w16 = plsc.bitcast(ering[e8, 2, pl.ds(g * _L, _L)], jnp.float32)
            for e in range(_L):
                wv = w16[e]
                er = g * _L + e
                for k4 in range(_DIM // _L):
                    sl = pl.ds(k4 * _L, _L)
                    rows[bb, er, sl] = rows[bb, er, sl] * wv

        pltpu.async_copy(rows.at[bb], acc.at[sidx.at[bb]], ssem.at[bb],
                         add=True)
        return 0

    lax.fori_loop(0, _CPT, chunk_body, 0)
    for t in range(_CPT - 3, _CPT):
        bb = t % _NR
        pltpu.make_async_copy(rows.at[bb], acc.at[sidx.at[bb]],
                              ssem.at[bb]).wait()


def _prop_body(emat, edata, out, acc, ering, rows, sidx, cbun,
               esem, gsem, ssem, zsem, csem, wsem):
    c = lax.axis_index("c")
    s = lax.axis_index("s")
    _zero_acc(s, acc, cbun, zsem)
    plsc.subcore_barrier()
    _edge_pass(c, s, emat, edata, acc, ering, rows, sidx, esem, gsem, ssem)
    plsc.subcore_barrier()
    pltpu.sync_copy(acc.at[pl.ds(s * _ROWS_PT, _ROWS_PT)],
                    out.at[pl.ds(c * _HALF + s * _ROWS_PT, _ROWS_PT)])


def _combine_body(emat, edata, e0, out, acc, ering, rows, sidx, cbun,
                  esem, gsem, ssem, zsem, csem, wsem):
    c = lax.axis_index("c")
    s = lax.axis_index("s")
    _zero_acc(s, acc, cbun, zsem)
    plsc.subcore_barrier()
    _edge_pass(c, s, emat, edata, acc, ering, rows, sidx, esem, gsem, ssem)
    plsc.subcore_barrier()
    r0 = c * _HALF + s * _ROWS_PT
    l00 = s * _ROWS_PT
    third = jnp.float32(1.0 / 3.0)

    def cb(b, _):
        pltpu.sync_copy(e0.at[pl.ds(r0 + b * _WCH, _WCH)], cbun.at[0, 0])
        pltpu.sync_copy(emat.at[pl.ds(r0 + b * _WCH, _WCH)], cbun.at[0, 1])
        pltpu.sync_copy(acc.at[pl.ds(l00 + b * _WCH, _WCH)], cbun.at[0, 2])
        for i in range(_WCH):
            for k4 in range(_DIM // _L):
                sl = pl.ds(k4 * _L, _L)
                cbun[0, 0, i, sl] = (cbun[0, 0, i, sl] + cbun[0, 1, i, sl]
                                     + cbun[0, 2, i, sl]) * third
        pltpu.sync_copy(cbun.at[0, 0], out.at[pl.ds(r0 + b * _WCH, _WCH)])
        return 0

    lax.fori_loop(0, _NWCH, cb, 0)


_SCRATCH = [
    pltpu.VMEM_SHARED((_HALF + 8, _DIM), jnp.float32),
    pltpu.VMEM((_NE, 3, _K), jnp.int32),
    pltpu.VMEM((_NR, _K, _DIM), jnp.float32),
    pltpu.VMEM((_NR, _K), jnp.int32),
    pltpu.VMEM((2, 3, _WCH, _DIM), jnp.float32),
    pltpu.SemaphoreType.DMA((_NE,)),
    pltpu.SemaphoreType.DMA((_NR,)),
    pltpu.SemaphoreType.DMA((_NR,)),
    pltpu.SemaphoreType.DMA,
    pltpu.SemaphoreType.DMA((2,)),
    pltpu.SemaphoreType.DMA((2,)),
]

_MESH = plsc.VectorSubcoreMesh(core_axis_name="c", subcore_axis_name="s")
_OUT = jax.ShapeDtypeStruct((_N_PAD, _DIM), jnp.float32)
_PARAMS = pltpu.CompilerParams(use_tc_tiling_on_sc=False,
                               needs_layout_passes=False)

_prop = pl.kernel(_prop_body, out_type=_OUT, mesh=_MESH,
                  scratch_types=_SCRATCH, compiler_params=_PARAMS,
                  name="lightgcn_prop")
_combine = pl.kernel(_combine_body, out_type=_OUT, mesh=_MESH,
                     scratch_types=_SCRATCH, compiler_params=_PARAMS,
                     name="lightgcn_prop_combine")


def kernel(edge_index, edge_weight, user_emb, item_emb):
    emat0 = jnp.concatenate(
        [user_emb, item_emb,
         jnp.zeros((_N_PAD - _N, _DIM), jnp.float32)], axis=0)
    dst = edge_index[0].astype(jnp.int32)
    col = edge_index[1].astype(jnp.int32)
    padn = _E_PAD - _E
    col_p = jnp.concatenate([col, jnp.zeros((padn,), jnp.int32)])
    dst_p = jnp.concatenate([dst, jnp.full((padn,), _N_PAD - 1, jnp.int32)])
    w_p = jnp.concatenate([edge_weight, jnp.zeros((padn,), jnp.float32)])
    edata = jnp.stack(
        [col_p.reshape(_E_PAD // _K, _K),
         dst_p.reshape(_E_PAD // _K, _K),
         jax.lax.bitcast_convert_type(w_p, jnp.int32).reshape(
             _E_PAD // _K, _K)], axis=1)
    e1 = _prop(emat0, edata)
    o = _combine(e1, edata, emat0)
    return o[:_NUM_USERS], o[_NUM_USERS:_N]
